# baseline (speedup 1.0000x reference)
import jax
import jax.numpy as jnp
from jax.experimental import pallas as pl
from jax.experimental.pallas import tpu as pltpu

_INDEX = 10
_BAND = 8
_BAND_BLK = _INDEX // _BAND
_BAND_OFF = _INDEX % _BAND
_BLOCK_B = 256


def _slice_body(x0_ref, x1_ref, o0_ref, o1_ref):
    o0_ref[...] = x0_ref[:, _BAND_OFF, :]
    o1_ref[...] = x1_ref[:, _BAND_OFF, :]


def kernel(x0, x1):
    B, S, D = x0.shape
    grid = (B // _BLOCK_B,)
    in_spec = pl.BlockSpec((_BLOCK_B, _BAND, D), lambda i: (i, _BAND_BLK, 0))
    out_spec = pl.BlockSpec((_BLOCK_B, D), lambda i: (i, 0))
    return pl.pallas_call(
        _slice_body,
        grid=grid,
        in_specs=[in_spec, in_spec],
        out_specs=[out_spec, out_spec],
        out_shape=[
            jax.ShapeDtypeStruct((B, D), x0.dtype),
            jax.ShapeDtypeStruct((B, D), x1.dtype),
        ],
        compiler_params=pltpu.CompilerParams(
            dimension_semantics=("arbitrary",),
        ),
    )(x0, x1)


# speedup vs baseline: 1.7760x; 1.7760x over previous
import jax
import jax.numpy as jnp
from jax.experimental import pallas as pl
from jax.experimental.pallas import tpu as pltpu

_INDEX = 10
_BLOCK_B = 512


def _slice_body(x0_ref, x1_ref, o0_ref, o1_ref):
    o0_ref[...] = x0_ref[0]
    o1_ref[...] = x1_ref[0]


def kernel(x0, x1):
    B, S, D = x0.shape
    x0t = jnp.transpose(x0, (1, 0, 2))
    x1t = jnp.transpose(x1, (1, 0, 2))
    grid = (B // _BLOCK_B,)
    in_spec = pl.BlockSpec((1, _BLOCK_B, D), lambda i: (_INDEX, i, 0))
    out_spec = pl.BlockSpec((_BLOCK_B, D), lambda i: (i, 0))
    return pl.pallas_call(
        _slice_body,
        grid=grid,
        in_specs=[in_spec, in_spec],
        out_specs=[out_spec, out_spec],
        out_shape=[
            jax.ShapeDtypeStruct((B, D), x0.dtype),
            jax.ShapeDtypeStruct((B, D), x1.dtype),
        ],
        compiler_params=pltpu.CompilerParams(
            dimension_semantics=("arbitrary",),
        ),
    )(x0t, x1t)


# speedup vs baseline: 91.8169x; 51.6980x over previous
import jax
import jax.numpy as jnp
from jax.experimental import pallas as pl
from jax.experimental.pallas import tpu as pltpu

_INDEX = 10
_BLOCK_B = 512


def _slice_body(x0_ref, x1_ref, o0_ref, o1_ref):
    o0_ref[...] = x0_ref[0]
    o1_ref[...] = x1_ref[0]


def kernel(x0, x1):
    B, S, D = x0.shape
    x0t = jnp.transpose(x0, (1, 2, 0))
    x1t = jnp.transpose(x1, (1, 2, 0))
    grid = (B // _BLOCK_B,)
    in_spec = pl.BlockSpec((1, D, _BLOCK_B), lambda i: (_INDEX, 0, i))
    out_spec = pl.BlockSpec((D, _BLOCK_B), lambda i: (0, i))
    o0t, o1t = pl.pallas_call(
        _slice_body,
        grid=grid,
        in_specs=[in_spec, in_spec],
        out_specs=[out_spec, out_spec],
        out_shape=[
            jax.ShapeDtypeStruct((D, B), x0.dtype),
            jax.ShapeDtypeStruct((D, B), x1.dtype),
        ],
        compiler_params=pltpu.CompilerParams(
            dimension_semantics=("arbitrary",),
        ),
    )(x0t, x1t)
    return jnp.transpose(o0t, (1, 0)), jnp.transpose(o1t, (1, 0))
